# baseline (speedup 1.0000x reference)
import jax
import jax.numpy as jnp
from jax import lax
from jax.experimental import pallas as pl
from jax.experimental.pallas import tpu as pltpu
from jax.experimental.pallas import tpu_sc as plsc

N = 20000
K = 117
TOPK = 100
NT = 16
CR = 128
NCHF = 9
SLOTS = 1280
LO = 0x3D4CCCCE
SH = 16
NBP = 768
SB = NBP // NT
CAP = 4096
GCAP = 4096
SENT = 0x7FFFFFFF
NEG = float("-inf")
I32 = jnp.int32
F32 = jnp.float32
TRUE16 = (True,) * 16


def _iota():
    return lax.iota(I32, 16)


def _si(x):
    return jnp.full((16,), x, I32)


def _sf(x):
    return jnp.full((16,), x, F32)


_GDN = lax.GatherDimensionNumbers(offset_dims=(), collapsed_slice_dims=(0,),
                                  start_index_map=(0,))


def _shuffle(v, idx):
    return lax.gather(v, idx[:, None], dimension_numbers=_GDN, slice_sizes=(1,),
                      mode=lax.GatherScatterMode.PROMISE_IN_BOUNDS)


def _lane_sum(v):
    for d in (8, 4, 2, 1):
        v = v + _shuffle(v, _iota() ^ d)
    return v


def _lane_max(v):
    for d in (8, 4, 2, 1):
        v = jnp.maximum(v, _shuffle(v, _iota() ^ d))
    return v


def _cnt(m):
    return jnp.sum(jnp.where(m, 1, 0).astype(I32))


def _body(hoi, psc, osc, pbox, obox, ocls,
          out_pb, out_ob, out_cls, out_act, out_sc,
          buf, grows, hist, histtot, bsv, ovec, rowmax, rowlist,
          candv, candi, gcandv, gcandi,
          redacc, redtmp, sufbuf, tvbuf, tmp16, tmp16b,
          outv, outidx, pairbuf, actbuf, scorebuf,
          pboxg, oboxg, clsg, idxcol, colbuf, id128, pg16, og16,
          sh_hist, sh_slicetot, sh_bthr, sh_candv, sh_candi,
          sh_outv, sh_outidx, counter, sem0, sem1, sem2):
    wid = lax.axis_index("s") + lax.axis_index("c") * NT
    lane = _iota()
    lane0 = lane == 0
    lane15 = lane == 15
    zi = _si(0)
    negv = _sf(NEG)
    all_true = lane >= 0

    @pl.when(wid == 0)
    def _():
        counter[0] = 0

    @plsc.parallel_loop(0, NBP, unroll=8)
    def _zero_hist(i):
        hist[pl.ds(i * 16, 16)] = zi
    for q in range(8):
        outv[pl.ds(q * 16, 16)] = zi
        outidx[pl.ds(q * 16, 16)] = zi
        id128[pl.ds(q * 16, 16)] = lane + q * 16
    for q in range(80):
        rowmax[pl.ds(q * 16, 16)] = negv
        rowlist[pl.ds(q * 16, 16)] = zi

    @pl.when(wid == 0)
    def _():
        tmp16[...] = zi
        for q in range(8):
            pltpu.sync_copy(tmp16, sh_outv.at[pl.ds(q * 16, 16)])
            pltpu.sync_copy(tmp16, sh_outidx.at[pl.ds(q * 16, 16)])

    def _coffr(ci):
        return pl.multiple_of(jnp.minimum(ci * CR, N - CR), 8)

    bs_cps = []
    for c in range(10):
        hoff = _coffr(wid + 16 * c)
        bs_cps.append(pltpu.async_copy(psc.at[pl.ds(hoff, CR)],
                                       bsv.at[pl.ds(c * CR, CR)], sem2))
        bs_cps.append(pltpu.async_copy(osc.at[pl.ds(hoff, CR)],
                                       ovec.at[pl.ds(c * CR, CR)], sem2))
    for d in bs_cps:
        d.wait()

    @plsc.parallel_loop(0, SLOTS // 16, unroll=4)
    def _bs(i):
        bsv[pl.ds(i * 16, 16)] = bsv[pl.ds(i * 16, 16)] * ovec[pl.ds(i * 16, 16)]

    lastcol = jnp.minimum(7 * 16 + lane, _si(K - 1))
    lastmask = (7 * 16 + lane) < _si(K)

    def _do_rows(bufc, cslot, ci):
        coffr = jnp.minimum(ci * CR, N - CR)

        @plsc.parallel_loop(0, CR, unroll=8)
        def _row(r):
            growr = coffr + r
            ok = (growr >= ci * CR) & (ci <= 156)
            okv = jnp.full((16,), ok)
            bs_s = plsc.load_gather(bsv, [_si(0) + cslot + r])
            acc = _sf(0.0)
            for jj in range(8):
                col = jj * 16 + lane if jj < 7 else lastcol
                v = plsc.load_gather(bufc, [_si(r), col])
                s = v * bs_s
                bits = plsc.bitcast(s, I32)
                m = (bits >= _si(LO)) & okv
                if jj == 7:
                    m = m & lastmask
                    s = jnp.where(lastmask, s, 0.0)
                idx = jnp.bitwise_and((bits - _si(LO)) >> (SH - 4), -16) | lane
                plsc.addupdate_scatter(hist, [idx], _si(1), mask=m)
                acc = jnp.maximum(acc, s)
            plsc.store_scatter(rowmax, [_si(0) + cslot + r], _lane_max(acc),
                               mask=lane0 & okv)

    def _chunk_src(ci):
        return hoi.at[pl.ds(_coffr(ci), CR)]

    pltpu.async_copy(_chunk_src(wid), buf.at[0], sem0)

    def _pair(p, c):
        c0 = wid + 16 * (2 * p)
        pltpu.make_async_copy(_chunk_src(c0), buf.at[0], sem0).wait()
        pltpu.async_copy(_chunk_src(c0 + 16), buf.at[1], sem1)
        _do_rows(buf.at[0], 2 * p * CR, c0)
        pltpu.make_async_copy(_chunk_src(c0), buf.at[1], sem1).wait()

        @pl.when(p < 4)
        def _():
            pltpu.async_copy(_chunk_src(c0 + 32), buf.at[0], sem0)
        _do_rows(buf.at[1], (2 * p + 1) * CR, c0 + 16)
        return c
    lax.fori_loop(0, 5, _pair, 0)

    @plsc.parallel_loop(0, NBP, unroll=4)
    def _lred(j):
        t = _lane_sum(hist[pl.ds(j * 16, 16)])
        plsc.store_scatter(histtot, [_si(j)], t, mask=lane0)
    pltpu.sync_copy(histtot, sh_hist.at[pl.ds(pl.multiple_of(wid * NBP, 8), NBP)])
    plsc.subcore_barrier()

    for q in range(SB // 16):
        redacc[pl.ds(q * 16, 16)] = zi

    def _slice_src(src):
        return sh_hist.at[pl.ds(pl.multiple_of(src * NBP + wid * SB, 8), SB)]

    pltpu.async_copy(_slice_src(0), redtmp.at[0], sem0)
    for src in range(NT):
        b = src % 2
        pltpu.make_async_copy(_slice_src(src), redtmp.at[b],
                              sem0 if b == 0 else sem1).wait()
        if src + 1 < NT:
            pltpu.async_copy(_slice_src(src + 1), redtmp.at[1 - b],
                             sem0 if (1 - b) == 0 else sem1)
        for q in range(SB // 16):
            redacc[pl.ds(q * 16, 16)] = (redacc[pl.ds(q * 16, 16)]
                                         + redtmp.at[b][pl.ds(q * 16, 16)])
    carry = zi
    for q in range(SB // 16 - 1, -1, -1):
        a = redacc[pl.ds(q * 16, 16)]
        suf = jnp.flip(plsc.cumsum(jnp.flip(a, axis=0)), axis=0) + carry
        sufbuf[pl.ds(q * 16, 16)] = suf
        carry = carry + _lane_sum(a)
    tmp16[...] = carry
    pltpu.sync_copy(tmp16, sh_slicetot.at[pl.ds(pl.multiple_of(wid * 16, 8), 16)])
    plsc.subcore_barrier()

    pltpu.sync_copy(sh_slicetot, tvbuf)
    totvec = plsc.load_gather(tvbuf, [lane * 16 + lane])
    above = jnp.flip(plsc.cumsum(jnp.flip(totvec, axis=0)), axis=0) - totvec
    ctot_v = _lane_sum(totvec)
    c_total = jnp.max(ctot_v)
    tmp16[...] = above
    above_s = plsc.load_gather(tmp16, [_si(0) + wid])
    tot_s = plsc.load_gather(tvbuf, [_si(wid * 16)])
    own_v = (above_s < _si(TOPK)) & ((above_s + tot_s) >= _si(TOPK))
    is_owner = jnp.max(jnp.where(own_v, 1, 0).astype(I32)) > 0

    @pl.when(is_owner)
    def _():
        ab = above_s
        ntrue = zi
        for q in range(SB // 16):
            suf = sufbuf[pl.ds(q * 16, 16)]
            ntrue = ntrue + jnp.where((suf + ab) >= _si(TOPK), 1, 0).astype(I32)
        nt_v = _lane_sum(ntrue)
        bglob = _si(wid * SB - 1) + nt_v
        tmp16b[...] = _si(LO) + (bglob << SH)
        pltpu.sync_copy(tmp16b, sh_bthr)

    plsc.subcore_barrier()
    pltpu.sync_copy(sh_bthr, tmp16)
    bthr_v = jnp.where(ctot_v < _si(TOPK), _si(LO), tmp16[...])
    sthr = plsc.bitcast(bthr_v, F32)

    def _scan_rows(q, off):
        rm = rowmax[pl.ds(q * 16, 16)]
        m = rm >= sthr
        slot = _si(q * 16) + lane
        civ = ((slot >> 7) << 4) + _si(0) + wid
        grow = jnp.minimum(civ * CR, _si(N - CR)) + (slot & _si(127))
        plsc.store_compressed(rowlist.at[pl.ds(off, 16)], grow, mask=m)
        return off + _cnt(m)
    cnt2 = lax.fori_loop(0, SLOTS // 16, _scan_rows, 0)

    def _grp_cond(st):
        g, coff = st
        return g * 16 < cnt2

    def _grp_body(st):
        g, coff = st
        rows16 = plsc.load_gather(rowlist, [_si(g * 16) + lane])
        pltpu.sync_copy(psc.at[rows16], pg16)
        pltpu.sync_copy(osc.at[rows16], og16)
        bs16 = pg16[...] * og16[...]

        def _slot(j, coff):
            validm = jnp.full((16,), g * 16 + j < cnt2)
            rsp = _shuffle(rows16, _si(0) + j)
            bs_s = _shuffle(bs16, _si(0) + j)
            row_s = jnp.max(rsp)
            roff = pl.multiple_of(jnp.bitwise_and(row_s, -8), 8)
            pltpu.sync_copy(hoi.at[pl.ds(roff, 8)], grows.at[pl.ds(0, 8)])
            rj = _si(0) + jnp.bitwise_and(row_s, 7)
            for jj in range(8):
                col = jj * 16 + lane
                colc = col if jj < 7 else lastcol
                v = plsc.load_gather(grows, [rj, colc])
                s = v * bs_s
                m = (s >= sthr) & validm
                if jj == 7:
                    m = m & lastmask
                plsc.store_compressed(candv.at[pl.ds(coff, 16)], s, mask=m)
                plsc.store_compressed(candi.at[pl.ds(coff, 16)], rsp * K + col,
                                      mask=m)
                coff = coff + _cnt(m)
            return coff
        coff = lax.fori_loop(0, 16, _slot, coff)
        return g + 1, coff

    _, coff = lax.while_loop(_grp_cond, _grp_body, (0, cnt2 * 0))

    w0m = _si(0) + wid == _si(0)
    pltpu.sync_copy(hoi.at[pl.ds(0, 3)], grows.at[pl.ds(0, 3)])
    for r in range(3):
        bs_s = plsc.load_gather(bsv, [_si(r)])
        for jj in range(8):
            col = jj * 16 + lane
            colc = col if jj < 7 else lastcol
            v = plsc.load_gather(grows, [_si(r), colc])
            s = v * bs_s
            passing = plsc.bitcast(s, I32) >= _si(LO)
            flat = _si(r * K) + col
            m = (~passing) & (col < _si(K)) & (flat < _si(256)) & w0m
            plsc.store_compressed(candv.at[pl.ds(coff, 16)], negv, mask=m)
            plsc.store_compressed(candi.at[pl.ds(coff, 16)], flat, mask=m)
            coff = coff + _cnt(m)

    plsc.store_compressed(candv.at[pl.ds(coff, 16)], negv, mask=all_true)
    plsc.store_compressed(candi.at[pl.ds(coff, 16)], _si(SENT), mask=all_true)
    offp = jnp.bitwise_and(coff + 15, -16)

    base = plsc.fetch_and_add(counter.at[0], offp, subcore_id=0)

    def _cp_cond(q):
        return q * 16 < offp

    def _cp_body(q):
        dst = pl.multiple_of(base + q * 16, 8)
        src16 = pl.multiple_of(q * 16, 8)
        pltpu.sync_copy(candv.at[pl.ds(src16, 16)],
                        sh_candv.at[pl.ds(dst, 16)])
        pltpu.sync_copy(candi.at[pl.ds(src16, 16)],
                        sh_candi.at[pl.ds(dst, 16)])
        return q + 1
    lax.while_loop(_cp_cond, _cp_body, 0)
    plsc.subcore_barrier()
    nn = plsc.fetch_and_add(counter.at[0], 0, subcore_id=0)

    def _pl_cond(kq):
        return kq * 256 < nn

    def _pl_body(kq):
        o = pl.multiple_of(kq * 256, 8)
        pltpu.sync_copy(sh_candv.at[pl.ds(o, 256)], gcandv.at[pl.ds(o, 256)])
        pltpu.sync_copy(sh_candi.at[pl.ds(o, 256)], gcandi.at[pl.ds(o, 256)])
        return kq + 1
    lax.while_loop(_pl_cond, _pl_body, 0)

    nv = nn // 16

    def _rank_one(t, c):
        j = wid + 16 * t
        vc = plsc.load_gather(gcandv, [_si(0) + j])
        ic = plsc.load_gather(gcandi, [_si(0) + j])

        def _rk(k, rkv):
            av = plsc.load_gather(gcandv, [_si(0) + k * 16 + lane])
            ai = plsc.load_gather(gcandi, [_si(0) + k * 16 + lane])
            better = (av > vc) | ((av == vc) & (ai < ic))
            return rkv + jnp.where(better, 1, 0).astype(I32)
        rkv = lax.fori_loop(0, nv, _rk, zi)
        rank = jnp.sum(rkv)
        wm = lane0 & jnp.full((16,), rank < TOPK)
        fv = jnp.where(vc == negv, _sf(0.0), vc)
        rix = _si(0) + jnp.minimum(rank, 127)
        plsc.store_scatter(outv, [rix], plsc.bitcast(fv, I32), mask=wm)
        plsc.store_scatter(outidx, [rix], ic, mask=wm)
        return c
    lax.fori_loop(0, nv, _rank_one, 0)

    pltpu.sync_copy(outv, sh_outv.at[id128], add=True)
    pltpu.sync_copy(outidx, sh_outidx.at[id128], add=True)
    plsc.subcore_barrier()

    @pl.when(wid == 0)
    def _():
        pltpu.sync_copy(sh_outv, outv)
        pltpu.sync_copy(sh_outidx, outidx)
        for q in range(8):
            ov = outv[pl.ds(q * 16, 16)]
            oi = outidx[pl.ds(q * 16, 16)]
            pv = lax.div(oi, _si(K))
            scorebuf[pl.ds(q * 16, 16)] = plsc.bitcast(ov, F32)
            pairbuf[pl.ds(q * 16, 16)] = pv
            actbuf[pl.ds(q * 16, 16)] = oi - pv * K
        for jcol in range(4):
            for q in range(8):
                idxcol[pl.ds(q * 16, 16)] = pairbuf[pl.ds(q * 16, 16)] * 4 + jcol
            pltpu.sync_copy(pbox.at[idxcol], colbuf)
            for q in range(8):
                plsc.store_scatter(pboxg, [lane + q * 16, _si(jcol)],
                                   colbuf[pl.ds(q * 16, 16)])
            pltpu.sync_copy(obox.at[idxcol], colbuf)
            for q in range(8):
                plsc.store_scatter(oboxg, [lane + q * 16, _si(jcol)],
                                   colbuf[pl.ds(q * 16, 16)])
        pltpu.sync_copy(ocls.at[pairbuf], idxcol)
        for q in range(8):
            plsc.store_scatter(clsg, [lane + q * 16, _si(0)],
                               idxcol[pl.ds(q * 16, 16)])
        pltpu.sync_copy(pboxg.at[pl.ds(0, TOPK)], out_pb)
        pltpu.sync_copy(oboxg.at[pl.ds(0, TOPK)], out_ob)
        pltpu.sync_copy(clsg.at[pl.ds(0, TOPK)], out_cls)
        pltpu.sync_copy(actbuf.at[pl.ds(0, TOPK)], out_act)
        pltpu.sync_copy(scorebuf.at[pl.ds(0, TOPK)], out_sc)


_MESH = plsc.VectorSubcoreMesh(core_axis_name="c", subcore_axis_name="s",
                               num_cores=1, num_subcores=NT)

_OUT_TYPE = (
    jax.ShapeDtypeStruct((TOPK, 4), F32),
    jax.ShapeDtypeStruct((TOPK, 4), F32),
    jax.ShapeDtypeStruct((TOPK, 1), I32),
    jax.ShapeDtypeStruct((TOPK,), I32),
    jax.ShapeDtypeStruct((TOPK,), F32),
)

_SCRATCH = (
    pltpu.VMEM((2, CR, K), F32),
    pltpu.VMEM((16, K), F32),
    pltpu.VMEM((NBP * 16,), I32),
    pltpu.VMEM((NBP,), I32),
    pltpu.VMEM((1280,), F32),
    pltpu.VMEM((1280,), F32),
    pltpu.VMEM((1280,), F32),
    pltpu.VMEM((1280,), I32),
    pltpu.VMEM((CAP,), F32),
    pltpu.VMEM((CAP,), I32),
    pltpu.VMEM((GCAP,), F32),
    pltpu.VMEM((GCAP,), I32),
    pltpu.VMEM((SB,), I32),
    pltpu.VMEM((2, SB), I32),
    pltpu.VMEM((SB,), I32),
    pltpu.VMEM((256,), I32),
    pltpu.VMEM((16,), I32),
    pltpu.VMEM((16,), I32),
    pltpu.VMEM((128,), I32),
    pltpu.VMEM((128,), I32),
    pltpu.VMEM((128,), I32),
    pltpu.VMEM((128,), I32),
    pltpu.VMEM((128,), F32),
    pltpu.VMEM((128, 4), F32),
    pltpu.VMEM((128, 4), F32),
    pltpu.VMEM((128, 1), I32),
    pltpu.VMEM((128,), I32),
    pltpu.VMEM((128,), F32),
    pltpu.VMEM((128,), I32),
    pltpu.VMEM((16,), F32),
    pltpu.VMEM((16,), F32),
    pltpu.VMEM_SHARED((NT * NBP,), I32),
    pltpu.VMEM_SHARED((256,), I32),
    pltpu.VMEM_SHARED((16,), I32),
    pltpu.VMEM_SHARED((GCAP,), F32),
    pltpu.VMEM_SHARED((GCAP,), I32),
    pltpu.VMEM_SHARED((128,), I32),
    pltpu.VMEM_SHARED((128,), I32),
    pltpu.SMEM((1,), I32),
    pltpu.SemaphoreType.DMA,
    pltpu.SemaphoreType.DMA,
    pltpu.SemaphoreType.DMA,
)

_sc_call = pl.kernel(_body, out_type=_OUT_TYPE, mesh=_MESH,
                     scratch_types=_SCRATCH,
                     compiler_params=pltpu.CompilerParams(
                         needs_layout_passes=False))


@jax.jit
def _run(person_boxes, object_boxes, person_box_scores, object_box_scores,
         classes_i32, hoi_scores):
    return _sc_call(hoi_scores, person_box_scores, object_box_scores,
                    person_boxes.reshape(-1), object_boxes.reshape(-1),
                    classes_i32.reshape(-1))


def kernel(person_boxes, object_boxes, person_box_scores, object_box_scores,
           object_box_classes, hoi_scores):
    cls32 = object_box_classes.astype(jnp.int32)
    return _run(person_boxes, object_boxes, person_box_scores,
                object_box_scores, cls32, hoi_scores)


# speedup vs baseline: 1.0647x; 1.0647x over previous
import jax
import jax.numpy as jnp
from jax import lax
from jax.experimental import pallas as pl
from jax.experimental.pallas import tpu as pltpu
from jax.experimental.pallas import tpu_sc as plsc

N = 20000
K = 117
TOPK = 100
NT = 16
CR = 128
NCHF = 9
SLOTS = 1280
LO = 0x3D4CCCCE
SH = 16
NBP = 768
SB = NBP // NT
CAP = 4096
GCAP = 4096
SENT = 0x7FFFFFFF
NEG = float("-inf")
I32 = jnp.int32
F32 = jnp.float32
TRUE16 = (True,) * 16


def _iota():
    return lax.iota(I32, 16)


def _si(x):
    return jnp.full((16,), x, I32)


def _sf(x):
    return jnp.full((16,), x, F32)


_GDN = lax.GatherDimensionNumbers(offset_dims=(), collapsed_slice_dims=(0,),
                                  start_index_map=(0,))


def _shuffle(v, idx):
    return lax.gather(v, idx[:, None], dimension_numbers=_GDN, slice_sizes=(1,),
                      mode=lax.GatherScatterMode.PROMISE_IN_BOUNDS)


def _lane_sum(v):
    for d in (8, 4, 2, 1):
        v = v + _shuffle(v, _iota() ^ d)
    return v


def _lane_max(v):
    for d in (8, 4, 2, 1):
        v = jnp.maximum(v, _shuffle(v, _iota() ^ d))
    return v


def _cnt(m):
    return jnp.sum(jnp.where(m, 1, 0).astype(I32))


def _body(hoi, psc, osc, pbox, obox, ocls,
          out_pb, out_ob, out_cls, out_act, out_sc,
          buf, grows, hist, histtot, bsv, ovec, rowmax, rowlist,
          candv, candi, gcandv, gcandi,
          redacc, redtmp, sufbuf, tvbuf, tmp16, tmp16b,
          outv, outidx, pairbuf, actbuf, scorebuf,
          pboxg, oboxg, clsg, idxcol, colbuf, id128, pg16, og16,
          sh_hist, sh_slicetot, sh_bthr, sh_candv, sh_candi,
          sh_outv, sh_outidx, counter, sem0, sem1, sem2):
    wid = lax.axis_index("s") + lax.axis_index("c") * NT
    lane = _iota()
    lane0 = lane == 0
    lane15 = lane == 15
    zi = _si(0)
    negv = _sf(NEG)
    all_true = lane >= 0

    @pl.when(wid == 0)
    def _():
        counter[0] = 0

    @plsc.parallel_loop(0, NBP, unroll=8)
    def _zero_hist(i):
        hist[pl.ds(i * 16, 16)] = zi
    for q in range(8):
        outv[pl.ds(q * 16, 16)] = zi
        outidx[pl.ds(q * 16, 16)] = zi
        id128[pl.ds(q * 16, 16)] = lane + q * 16
    for q in range(80):
        rowmax[pl.ds(q * 16, 16)] = negv
        rowlist[pl.ds(q * 16, 16)] = zi

    @pl.when(wid == 0)
    def _():
        tmp16[...] = zi
        for q in range(8):
            pltpu.sync_copy(tmp16, sh_outv.at[pl.ds(q * 16, 16)])
            pltpu.sync_copy(tmp16, sh_outidx.at[pl.ds(q * 16, 16)])

    def _coffr(ci):
        return pl.multiple_of(jnp.minimum(ci * CR, N - CR), 8)

    bs_cps = []
    for c in range(10):
        hoff = _coffr(wid + 16 * c)
        bs_cps.append(pltpu.async_copy(psc.at[pl.ds(hoff, CR)],
                                       bsv.at[pl.ds(c * CR, CR)], sem2))
        bs_cps.append(pltpu.async_copy(osc.at[pl.ds(hoff, CR)],
                                       ovec.at[pl.ds(c * CR, CR)], sem2))
    for d in bs_cps:
        d.wait()

    @plsc.parallel_loop(0, SLOTS // 16, unroll=4)
    def _bs(i):
        bsv[pl.ds(i * 16, 16)] = bsv[pl.ds(i * 16, 16)] * ovec[pl.ds(i * 16, 16)]

    lastcol = jnp.minimum(7 * 16 + lane, _si(K - 1))
    lastmask = (7 * 16 + lane) < _si(K)

    def _do_rows(bufc, cslot, ci):
        coffr = jnp.minimum(ci * CR, N - CR)

        @plsc.parallel_loop(0, CR, unroll=4)
        def _row(r):
            growr = coffr + r
            ok = (growr >= ci * CR) & (ci <= 156)
            okv = jnp.full((16,), ok)
            bs_s = plsc.load_gather(bsv, [_si(0) + cslot + r])
            acc = _sf(0.0)
            for jj in range(8):
                col = jj * 16 + lane if jj < 7 else lastcol
                v = plsc.load_gather(bufc, [_si(r), col])
                s = v * bs_s
                bits = plsc.bitcast(s, I32)
                m = (bits >= _si(LO)) & okv
                if jj == 7:
                    m = m & lastmask
                    s = jnp.where(lastmask, s, 0.0)
                idx = jnp.bitwise_and((bits - _si(LO)) >> (SH - 4), -16) | lane
                plsc.addupdate_scatter(hist, [idx], _si(1), mask=m)
                acc = jnp.maximum(acc, s)
            plsc.store_scatter(rowmax, [_si(0) + cslot + r], _lane_max(acc),
                               mask=lane0 & okv)

    def _chunk_src(ci):
        return hoi.at[pl.ds(_coffr(ci), CR)]

    pltpu.async_copy(_chunk_src(wid), buf.at[0], sem0)

    def _pair(p, c):
        c0 = wid + 16 * (2 * p)
        pltpu.make_async_copy(_chunk_src(c0), buf.at[0], sem0).wait()
        pltpu.async_copy(_chunk_src(c0 + 16), buf.at[1], sem1)
        _do_rows(buf.at[0], 2 * p * CR, c0)
        pltpu.make_async_copy(_chunk_src(c0), buf.at[1], sem1).wait()

        @pl.when(p < 4)
        def _():
            pltpu.async_copy(_chunk_src(c0 + 32), buf.at[0], sem0)
        _do_rows(buf.at[1], (2 * p + 1) * CR, c0 + 16)
        return c
    lax.fori_loop(0, 5, _pair, 0)

    @plsc.parallel_loop(0, NBP, unroll=4)
    def _lred(j):
        t = _lane_sum(hist[pl.ds(j * 16, 16)])
        plsc.store_scatter(histtot, [_si(j)], t, mask=lane0)
    pltpu.sync_copy(histtot, sh_hist.at[pl.ds(pl.multiple_of(wid * NBP, 8), NBP)])
    plsc.subcore_barrier()

    for q in range(SB // 16):
        redacc[pl.ds(q * 16, 16)] = zi

    def _slice_src(src):
        return sh_hist.at[pl.ds(pl.multiple_of(src * NBP + wid * SB, 8), SB)]

    pltpu.async_copy(_slice_src(0), redtmp.at[0], sem0)
    for src in range(NT):
        b = src % 2
        pltpu.make_async_copy(_slice_src(src), redtmp.at[b],
                              sem0 if b == 0 else sem1).wait()
        if src + 1 < NT:
            pltpu.async_copy(_slice_src(src + 1), redtmp.at[1 - b],
                             sem0 if (1 - b) == 0 else sem1)
        for q in range(SB // 16):
            redacc[pl.ds(q * 16, 16)] = (redacc[pl.ds(q * 16, 16)]
                                         + redtmp.at[b][pl.ds(q * 16, 16)])
    carry = zi
    for q in range(SB // 16 - 1, -1, -1):
        a = redacc[pl.ds(q * 16, 16)]
        suf = jnp.flip(plsc.cumsum(jnp.flip(a, axis=0)), axis=0) + carry
        sufbuf[pl.ds(q * 16, 16)] = suf
        carry = carry + _lane_sum(a)
    tmp16[...] = carry
    pltpu.sync_copy(tmp16, sh_slicetot.at[pl.ds(pl.multiple_of(wid * 16, 8), 16)])
    plsc.subcore_barrier()

    pltpu.sync_copy(sh_slicetot, tvbuf)
    totvec = plsc.load_gather(tvbuf, [lane * 16 + lane])
    above = jnp.flip(plsc.cumsum(jnp.flip(totvec, axis=0)), axis=0) - totvec
    ctot_v = _lane_sum(totvec)
    c_total = jnp.max(ctot_v)
    tmp16[...] = above
    above_s = plsc.load_gather(tmp16, [_si(0) + wid])
    tot_s = plsc.load_gather(tvbuf, [_si(wid * 16)])
    own_v = (above_s < _si(TOPK)) & ((above_s + tot_s) >= _si(TOPK))
    is_owner = jnp.max(jnp.where(own_v, 1, 0).astype(I32)) > 0

    @pl.when(is_owner)
    def _():
        ab = above_s
        ntrue = zi
        for q in range(SB // 16):
            suf = sufbuf[pl.ds(q * 16, 16)]
            ntrue = ntrue + jnp.where((suf + ab) >= _si(TOPK), 1, 0).astype(I32)
        nt_v = _lane_sum(ntrue)
        bglob = _si(wid * SB - 1) + nt_v
        tmp16b[...] = _si(LO) + (bglob << SH)
        pltpu.sync_copy(tmp16b, sh_bthr)

    plsc.subcore_barrier()
    pltpu.sync_copy(sh_bthr, tmp16)
    bthr_v = jnp.where(ctot_v < _si(TOPK), _si(LO), tmp16[...])
    sthr = plsc.bitcast(bthr_v, F32)

    def _scan_rows(q, off):
        rm = rowmax[pl.ds(q * 16, 16)]
        m = rm >= sthr
        slot = _si(q * 16) + lane
        civ = ((slot >> 7) << 4) + _si(0) + wid
        grow = jnp.minimum(civ * CR, _si(N - CR)) + (slot & _si(127))
        plsc.store_compressed(rowlist.at[pl.ds(off, 16)], grow, mask=m)
        return off + _cnt(m)
    cnt2 = lax.fori_loop(0, SLOTS // 16, _scan_rows, 0)

    def _grp_cond(st):
        g, coff = st
        return g * 16 < cnt2

    def _grp_body(st):
        g, coff = st
        rows16 = plsc.load_gather(rowlist, [_si(g * 16) + lane])
        pltpu.sync_copy(psc.at[rows16], pg16)
        pltpu.sync_copy(osc.at[rows16], og16)
        bs16 = pg16[...] * og16[...]

        def _slot(j, coff):
            validm = jnp.full((16,), g * 16 + j < cnt2)
            rsp = _shuffle(rows16, _si(0) + j)
            bs_s = _shuffle(bs16, _si(0) + j)
            row_s = jnp.max(rsp)
            roff = pl.multiple_of(jnp.bitwise_and(row_s, -8), 8)
            pltpu.sync_copy(hoi.at[pl.ds(roff, 8)], grows.at[pl.ds(0, 8)])
            rj = _si(0) + jnp.bitwise_and(row_s, 7)
            for jj in range(8):
                col = jj * 16 + lane
                colc = col if jj < 7 else lastcol
                v = plsc.load_gather(grows, [rj, colc])
                s = v * bs_s
                m = (s >= sthr) & validm
                if jj == 7:
                    m = m & lastmask
                plsc.store_compressed(candv.at[pl.ds(coff, 16)], s, mask=m)
                plsc.store_compressed(candi.at[pl.ds(coff, 16)], rsp * K + col,
                                      mask=m)
                coff = coff + _cnt(m)
            return coff
        coff = lax.fori_loop(0, 16, _slot, coff)
        return g + 1, coff

    _, coff = lax.while_loop(_grp_cond, _grp_body, (0, cnt2 * 0))

    w0m = _si(0) + wid == _si(0)
    pltpu.sync_copy(hoi.at[pl.ds(0, 3)], grows.at[pl.ds(0, 3)])
    for r in range(3):
        bs_s = plsc.load_gather(bsv, [_si(r)])
        for jj in range(8):
            col = jj * 16 + lane
            colc = col if jj < 7 else lastcol
            v = plsc.load_gather(grows, [_si(r), colc])
            s = v * bs_s
            passing = plsc.bitcast(s, I32) >= _si(LO)
            flat = _si(r * K) + col
            m = (~passing) & (col < _si(K)) & (flat < _si(256)) & w0m
            plsc.store_compressed(candv.at[pl.ds(coff, 16)], negv, mask=m)
            plsc.store_compressed(candi.at[pl.ds(coff, 16)], flat, mask=m)
            coff = coff + _cnt(m)

    plsc.store_compressed(candv.at[pl.ds(coff, 16)], negv, mask=all_true)
    plsc.store_compressed(candi.at[pl.ds(coff, 16)], _si(SENT), mask=all_true)
    offp = jnp.bitwise_and(coff + 15, -16)

    base = plsc.fetch_and_add(counter.at[0], offp, subcore_id=0)

    def _cp_cond(q):
        return q * 16 < offp

    def _cp_body(q):
        dst = pl.multiple_of(base + q * 16, 8)
        src16 = pl.multiple_of(q * 16, 8)
        pltpu.sync_copy(candv.at[pl.ds(src16, 16)],
                        sh_candv.at[pl.ds(dst, 16)])
        pltpu.sync_copy(candi.at[pl.ds(src16, 16)],
                        sh_candi.at[pl.ds(dst, 16)])
        return q + 1
    lax.while_loop(_cp_cond, _cp_body, 0)
    plsc.subcore_barrier()
    nn = plsc.fetch_and_add(counter.at[0], 0, subcore_id=0)

    def _pl_cond(kq):
        return kq * 256 < nn

    def _pl_body(kq):
        o = pl.multiple_of(kq * 256, 8)
        pltpu.sync_copy(sh_candv.at[pl.ds(o, 256)], gcandv.at[pl.ds(o, 256)])
        pltpu.sync_copy(sh_candi.at[pl.ds(o, 256)], gcandi.at[pl.ds(o, 256)])
        return kq + 1
    lax.while_loop(_pl_cond, _pl_body, 0)

    nv = nn // 16

    def _rank_one(t, c):
        j = wid + 16 * t
        vc = plsc.load_gather(gcandv, [_si(0) + j])
        ic = plsc.load_gather(gcandi, [_si(0) + j])

        def _rk(k, rkv):
            av = plsc.load_gather(gcandv, [_si(0) + k * 16 + lane])
            ai = plsc.load_gather(gcandi, [_si(0) + k * 16 + lane])
            better = (av > vc) | ((av == vc) & (ai < ic))
            return rkv + jnp.where(better, 1, 0).astype(I32)
        rkv = lax.fori_loop(0, nv, _rk, zi)
        rank = jnp.sum(rkv)
        wm = lane0 & jnp.full((16,), rank < TOPK)
        fv = jnp.where(vc == negv, _sf(0.0), vc)
        rix = _si(0) + jnp.minimum(rank, 127)
        plsc.store_scatter(outv, [rix], plsc.bitcast(fv, I32), mask=wm)
        plsc.store_scatter(outidx, [rix], ic, mask=wm)
        return c
    lax.fori_loop(0, nv, _rank_one, 0)

    pltpu.sync_copy(outv, sh_outv.at[id128], add=True)
    pltpu.sync_copy(outidx, sh_outidx.at[id128], add=True)
    plsc.subcore_barrier()

    @pl.when(wid == 0)
    def _():
        pltpu.sync_copy(sh_outv, outv)
        pltpu.sync_copy(sh_outidx, outidx)
        for q in range(8):
            ov = outv[pl.ds(q * 16, 16)]
            oi = outidx[pl.ds(q * 16, 16)]
            pv = lax.div(oi, _si(K))
            scorebuf[pl.ds(q * 16, 16)] = plsc.bitcast(ov, F32)
            pairbuf[pl.ds(q * 16, 16)] = pv
            actbuf[pl.ds(q * 16, 16)] = oi - pv * K
        for jcol in range(4):
            for q in range(8):
                idxcol[pl.ds(q * 16, 16)] = pairbuf[pl.ds(q * 16, 16)] * 4 + jcol
            pltpu.sync_copy(pbox.at[idxcol], colbuf)
            for q in range(8):
                plsc.store_scatter(pboxg, [lane + q * 16, _si(jcol)],
                                   colbuf[pl.ds(q * 16, 16)])
            pltpu.sync_copy(obox.at[idxcol], colbuf)
            for q in range(8):
                plsc.store_scatter(oboxg, [lane + q * 16, _si(jcol)],
                                   colbuf[pl.ds(q * 16, 16)])
        pltpu.sync_copy(ocls.at[pairbuf], idxcol)
        for q in range(8):
            plsc.store_scatter(clsg, [lane + q * 16, _si(0)],
                               idxcol[pl.ds(q * 16, 16)])
        pltpu.sync_copy(pboxg.at[pl.ds(0, TOPK)], out_pb)
        pltpu.sync_copy(oboxg.at[pl.ds(0, TOPK)], out_ob)
        pltpu.sync_copy(clsg.at[pl.ds(0, TOPK)], out_cls)
        pltpu.sync_copy(actbuf.at[pl.ds(0, TOPK)], out_act)
        pltpu.sync_copy(scorebuf.at[pl.ds(0, TOPK)], out_sc)


_MESH = plsc.VectorSubcoreMesh(core_axis_name="c", subcore_axis_name="s",
                               num_cores=1, num_subcores=NT)

_OUT_TYPE = (
    jax.ShapeDtypeStruct((TOPK, 4), F32),
    jax.ShapeDtypeStruct((TOPK, 4), F32),
    jax.ShapeDtypeStruct((TOPK, 1), I32),
    jax.ShapeDtypeStruct((TOPK,), I32),
    jax.ShapeDtypeStruct((TOPK,), F32),
)

_SCRATCH = (
    pltpu.VMEM((2, CR, K), F32),
    pltpu.VMEM((16, K), F32),
    pltpu.VMEM((NBP * 16,), I32),
    pltpu.VMEM((NBP,), I32),
    pltpu.VMEM((1280,), F32),
    pltpu.VMEM((1280,), F32),
    pltpu.VMEM((1280,), F32),
    pltpu.VMEM((1280,), I32),
    pltpu.VMEM((CAP,), F32),
    pltpu.VMEM((CAP,), I32),
    pltpu.VMEM((GCAP,), F32),
    pltpu.VMEM((GCAP,), I32),
    pltpu.VMEM((SB,), I32),
    pltpu.VMEM((2, SB), I32),
    pltpu.VMEM((SB,), I32),
    pltpu.VMEM((256,), I32),
    pltpu.VMEM((16,), I32),
    pltpu.VMEM((16,), I32),
    pltpu.VMEM((128,), I32),
    pltpu.VMEM((128,), I32),
    pltpu.VMEM((128,), I32),
    pltpu.VMEM((128,), I32),
    pltpu.VMEM((128,), F32),
    pltpu.VMEM((128, 4), F32),
    pltpu.VMEM((128, 4), F32),
    pltpu.VMEM((128, 1), I32),
    pltpu.VMEM((128,), I32),
    pltpu.VMEM((128,), F32),
    pltpu.VMEM((128,), I32),
    pltpu.VMEM((16,), F32),
    pltpu.VMEM((16,), F32),
    pltpu.VMEM_SHARED((NT * NBP,), I32),
    pltpu.VMEM_SHARED((256,), I32),
    pltpu.VMEM_SHARED((16,), I32),
    pltpu.VMEM_SHARED((GCAP,), F32),
    pltpu.VMEM_SHARED((GCAP,), I32),
    pltpu.VMEM_SHARED((128,), I32),
    pltpu.VMEM_SHARED((128,), I32),
    pltpu.SMEM((1,), I32),
    pltpu.SemaphoreType.DMA,
    pltpu.SemaphoreType.DMA,
    pltpu.SemaphoreType.DMA,
)

_sc_call = pl.kernel(_body, out_type=_OUT_TYPE, mesh=_MESH,
                     scratch_types=_SCRATCH,
                     compiler_params=pltpu.CompilerParams(
                         needs_layout_passes=False))


@jax.jit
def _run(person_boxes, object_boxes, person_box_scores, object_box_scores,
         classes_i32, hoi_scores):
    return _sc_call(hoi_scores, person_box_scores, object_box_scores,
                    person_boxes.reshape(-1), object_boxes.reshape(-1),
                    classes_i32.reshape(-1))


def kernel(person_boxes, object_boxes, person_box_scores, object_box_scores,
           object_box_classes, hoi_scores):
    cls32 = object_box_classes.astype(jnp.int32)
    return _run(person_boxes, object_boxes, person_box_scores,
                object_box_scores, cls32, hoi_scores)
